# baseline (speedup 1.0000x reference)
import functools

import jax
import jax.numpy as jnp
from jax import lax
from jax.experimental import pallas as pl
from jax.experimental.pallas import tpu as pltpu
from jax.experimental.pallas import tpu_sc as plsc

_B, _C = 2, 96
_OH = _OW = 27
_KH = _KW = 16
_H = _W = 224
_N_IMG = _B * _C
_ROW_ELEMS = _OW * _KH * _KW
_RROWS = _ROW_ELEMS // 128
_IMG_OUT = _H * _W
_N_WORKERS = 32
_IMGS_PER_WORKER = _N_IMG // _N_WORKERS
_XROWS = _N_IMG * _OH * _RROWS
_FETCH = 64


def _fold_sc(xr):
    mesh = plsc.VectorSubcoreMesh(core_axis_name="c", subcore_axis_name="s")

    @functools.partial(
        pl.kernel,
        out_type=jax.ShapeDtypeStruct((_N_IMG * _IMG_OUT,), jnp.float32),
        mesh=mesh,
        scratch_types=[
            pltpu.VMEM((_FETCH, 128), jnp.float32),
            pltpu.VMEM((_FETCH, 128), jnp.float32),
            pltpu.VMEM((_FETCH, 128), jnp.float32),
            pltpu.VMEM((_IMG_OUT,), jnp.float32),
            pltpu.VMEM((_IMG_OUT,), jnp.float32),
            pltpu.SemaphoreType.DMA,
            pltpu.SemaphoreType.DMA,
            pltpu.SemaphoreType.DMA,
            pltpu.SemaphoreType.DMA,
            pltpu.SemaphoreType.DMA,
        ],
    )
    def k(x_hbm, out_hbm, rb0, rb1, rb2, obA, obB, s0, s1, s2, soA, soB):
        wid = lax.axis_index("s") * 2 + lax.axis_index("c")
        zeros16 = jnp.zeros((16,), jnp.float32)
        rbufs = [rb0, rb1, rb2]
        sems = [s0, s1, s2]
        obufs = [obA, obB]
        osems = [soA, soB]

        def chunk_base(img, i):
            row0 = (img * _OH + i) * _RROWS
            base = jnp.minimum((row0 // 8) * 8, _XROWS - _FETCH)
            return pl.multiple_of(base, 8), row0 - base

        def start_in(img, i, slot):
            base, _ = chunk_base(img, i)
            pltpu.async_copy(
                x_hbm.at[pl.ds(base, _FETCH)], rbufs[slot], sems[slot]
            )

        def wait_in(slot):
            pltpu.make_async_copy(
                x_hbm.at[pl.ds(0, _FETCH)], rbufs[slot], sems[slot]
            ).wait()

        def wait_out(p):
            pltpu.make_async_copy(
                obufs[p], out_hbm.at[pl.ds(0, _IMG_OUT)], osems[p]
            ).wait()

        def run_image(img, p, first):
            obuf = obufs[p]
            start_in(img, 0, 0)
            start_in(img, 1, 1)
            if not first:
                wait_out(p)

            def zero_body(t, carry):
                for u in range(_W // 16):
                    obuf[pl.ds(t * _W + u * 16, 16)] = zeros16
                return carry

            lax.fori_loop(0, _H, zero_body, 0)

            def g_body(g, carry):
                for s in range(3):
                    i = 3 * g + s
                    wait_in(s)
                    nslot = (s + 2) % 3

                    @pl.when(i + 2 <= _OH - 1)
                    def _():
                        start_in(img, i + 2, nslot)

                    rbuf = rbufs[s]
                    _, shift = chunk_base(img, i)

                    def j_body(j, inner):
                        r0 = shift + 2 * j
                        vs = [
                            rbuf[r0 + (kh // 8), pl.ds((kh % 8) * 16, 16)]
                            for kh in range(_KH)
                        ]
                        dbase = (8 * i) * _W + 8 * j
                        for kh in range(_KH):
                            plsc.addupdate(
                                obuf.at[pl.ds(dbase + kh * _W, 16)], vs[kh]
                            )
                        return inner

                    lax.fori_loop(0, _OW, j_body, 0)
                return carry

            lax.fori_loop(0, _OH // 3, g_body, 0)
            pltpu.async_copy(
                obuf, out_hbm.at[pl.ds(img * _IMG_OUT, _IMG_OUT)], osems[p]
            )

        for m in range(_IMGS_PER_WORKER):
            run_image(wid * _IMGS_PER_WORKER + m, m % 2, m < 2)
        wait_out(0)
        wait_out(1)

    return k(xr)


def kernel(x):
    xr = x.reshape(_N_IMG * _OH * _RROWS, 128)
    out = _fold_sc(xr)
    return out.reshape(_B, _C, _H, _W)
